# baseline (speedup 1.0000x reference)
import functools

import jax
import jax.numpy as jnp
from jax import lax
from jax.experimental import pallas as pl
from jax.experimental.pallas import tpu as pltpu
from jax.experimental.pallas import tpu_sc as plsc

H = 128
NRBF = 20
CUT = 5.0
EPS = 1e-8
NG = 64
N_E = 320000
N_N = 10000
E_PAD = 327680
N_PAD = 10240

CH = 128
NROWS = E_PAD // CH
GW_ROWS = NROWS // 32
SC_ROWS = NROWS // 16
NPT = N_PAD // 16
BE = 512
BN = 1024


def _sc_gather(num_tables):
    mesh = plsc.VectorSubcoreMesh(core_axis_name="c", subcore_axis_name="s")

    @functools.partial(
        pl.kernel, mesh=mesh,
        out_type=[jax.ShapeDtypeStruct((E_PAD, H), jnp.float32)] * num_tables,
        scratch_types=[
            pltpu.VMEM((GW_ROWS, CH), jnp.int32),
            pltpu.VMEM((CH, H), jnp.float32),
            pltpu.SemaphoreType.DMA,
        ],
    )
    def k(*refs):
        tables = refs[:num_tables]
        idx_hbm = refs[num_tables]
        outs = refs[num_tables + 1: 2 * num_tables + 1]
        idx_v, rows_v, sem = refs[2 * num_tables + 1:]
        wid = lax.axis_index("s") * 2 + lax.axis_index("c")
        row0 = wid * GW_ROWS
        pltpu.sync_copy(idx_hbm.at[pl.ds(row0, GW_ROWS)], idx_v)
        for tb, ob in zip(tables, outs):
            def body(t, _, tb=tb, ob=ob):
                pltpu.async_copy(tb.at[idx_v.at[t]], rows_v, sem).wait()
                pltpu.sync_copy(rows_v, ob.at[pl.ds((row0 + t) * CH, CH)])
                return ()
            lax.fori_loop(0, GW_ROWS, body, ())

    return k


def _sc_scatter4():
    mesh = plsc.VectorSubcoreMesh(core_axis_name="c", subcore_axis_name="s")

    @functools.partial(
        pl.kernel, mesh=mesh,
        out_type=[jax.ShapeDtypeStruct((N_PAD, H), jnp.float32)] * 4,
        scratch_types=[
            pltpu.VMEM((SC_ROWS, CH), jnp.int32),
            pltpu.VMEM((CH, H), jnp.float32),
            pltpu.VMEM((32, H), jnp.float32),
            pltpu.VMEM_SHARED((N_PAD, H), jnp.float32),
            pltpu.SemaphoreType.DMA,
        ],
    )
    def k(v0, v1, v2, v3, idx_hbm, o0, o1, o2, o3,
          idx_v, rows_v, zbuf, acc, sem):
        c = lax.axis_index("c")
        s = lax.axis_index("s")
        row0 = s * SC_ROWS
        pltpu.sync_copy(idx_hbm.at[pl.ds(row0, SC_ROWS)], idx_v)

        def zfill(t, _):
            i = t // 8
            j = (t % 8) * 16
            zbuf[i, pl.ds(j, 16)] = jnp.zeros((16,), jnp.float32)
            return ()
        lax.fori_loop(0, 32 * 8, zfill, ())

        nbase = s * NPT

        def do_array(vals, out):
            def zb(t, _):
                pltpu.sync_copy(zbuf, acc.at[pl.ds(nbase + t * 32, 32)])
                return ()
            lax.fori_loop(0, NPT // 32, zb, ())
            plsc.subcore_barrier()

            def body(t, _, vals=vals):
                pltpu.sync_copy(vals.at[pl.ds((row0 + t) * CH, CH)], rows_v)
                pltpu.sync_copy(rows_v, acc.at[idx_v.at[t]], add=True)
                return ()
            lax.fori_loop(0, SC_ROWS, body, ())
            plsc.subcore_barrier()
            pltpu.sync_copy(acc.at[pl.ds(nbase, NPT)],
                            out.at[pl.ds(nbase, NPT)])

        @pl.when(c == 0)
        def _():
            do_array(v0, o0)
            do_array(v1, o1)

        @pl.when(c == 1)
        def _():
            do_array(v2, o2)
            do_array(v3, o3)

    return k


def _silu(x):
    return x * (1.0 / (1.0 + jnp.exp(-x)))


def _edge_body(has_v, refs):
    if has_v:
        (ef, sj, vj0, vj1, vj2, w1t, b1, w2t, b2, wst, bs, wvt, bv,
         ms_o, a0_o, a1_o, a2_o) = refs
    else:
        (ef, sj, w1t, b1, w2t, b2, wst, bs, wvt, bv,
         ms_o, a0_o, a1_o, a2_o) = refs
    e = ef[...]
    u0 = e[:, 0:1]
    u1 = e[:, 1:2]
    u2 = e[:, 2:3]
    r = e[:, 3:4]
    kcol = lax.broadcasted_iota(jnp.int32, (BE, H), 1).astype(jnp.float32)
    freqs = jnp.where(kcol < NRBF, (kcol + 1.0) * (jnp.pi / CUT), 0.0)
    cv = 0.5 * (jnp.cos(r * (jnp.pi / CUT)) + 1.0)
    cv = jnp.where(r < CUT, cv, 0.0)
    rbf = jnp.sin(r * freqs) / r * cv
    h1 = _silu(jnp.dot(rbf, w1t[...], preferred_element_type=jnp.float32)
               + b1[...])
    w = jnp.dot(h1, w2t[...], preferred_element_type=jnp.float32) + b2[...]
    ms = (jnp.dot(sj[...], wst[...], preferred_element_type=jnp.float32)
          + bs[...]) * w
    if has_v:
        proj = u0 * vj0[...] + u1 * vj1[...] + u2 * vj2[...]
        mv = (jnp.dot(proj, wvt[...], preferred_element_type=jnp.float32)
              + bv[...]) * w
    else:
        mv = bv[...] * w
    ms_o[...] = ms
    a0_o[...] = u0 * mv
    a1_o[...] = u1 * mv
    a2_o[...] = u2 * mv


def _tc_edge(has_v, interpret=False):
    nb = E_PAD // BE
    big = pl.BlockSpec((BE, H), lambda t: (t, 0))
    wspec = pl.BlockSpec((H, H), lambda t: (0, 0))
    bspec = pl.BlockSpec((1, H), lambda t: (0, 0))
    in_specs = [pl.BlockSpec((BE, 8), lambda t: (t, 0)), big]
    if has_v:
        in_specs += [big, big, big]
    in_specs += [wspec, bspec, wspec, bspec, wspec, bspec, wspec, bspec]
    return pl.pallas_call(
        lambda *refs: _edge_body(has_v, refs),
        grid=(nb,),
        in_specs=in_specs,
        out_specs=[big, big, big, big],
        out_shape=[jax.ShapeDtypeStruct((E_PAD, H), jnp.float32)] * 4,
        interpret=interpret,
    )


def _node_body(*refs):
    (s, ms, v0, v1, v2, a0, a1, a2,
     us1t, usb1, us2t, usb2, uv1t, uvb1, uv2t, uvb2,
     s_o, v0_o, v1_o, v2_o) = refs

    def mlp(x, w1t, bb1, w2t, bb2):
        h = _silu(jnp.dot(x, w1t[...], preferred_element_type=jnp.float32)
                  + bb1[...])
        return jnp.dot(h, w2t[...], preferred_element_type=jnp.float32) + bb2[...]

    s_o[...] = s[...] + mlp(ms[...], us1t, usb1, us2t, usb2)
    v0_o[...] = v0[...] + mlp(a0[...], uv1t, uvb1, uv2t, uvb2)
    v1_o[...] = v1[...] + mlp(a1[...], uv1t, uvb1, uv2t, uvb2)
    v2_o[...] = v2[...] + mlp(a2[...], uv1t, uvb1, uv2t, uvb2)


def _tc_node(interpret=False):
    nb = N_PAD // BN
    big = pl.BlockSpec((BN, H), lambda t: (t, 0))
    wspec = pl.BlockSpec((H, H), lambda t: (0, 0))
    bspec = pl.BlockSpec((1, H), lambda t: (0, 0))
    return pl.pallas_call(
        _node_body,
        grid=(nb,),
        in_specs=[big] * 8 + [wspec, bspec, wspec, bspec] * 2,
        out_specs=[big] * 4,
        out_shape=[jax.ShapeDtypeStruct((N_PAD, H), jnp.float32)] * 4,
        interpret=interpret,
    )


def _readout_body(s, batch, wrot, brow, out):
    @pl.when(pl.program_id(0) == 0)
    def _():
        out[...] = jnp.zeros_like(out)
    per_atom = (jnp.dot(s[...], wrot[...], preferred_element_type=jnp.float32)
                + brow[...])
    bvec = batch[0]
    gid = lax.broadcasted_iota(jnp.int32, (NG, BN), 0)
    onehot = (gid == bvec).astype(jnp.float32)
    out[...] += jnp.dot(onehot, per_atom, preferred_element_type=jnp.float32)


def _tc_readout(interpret=False):
    nb = N_PAD // BN
    return pl.pallas_call(
        _readout_body,
        grid=(nb,),
        in_specs=[
            pl.BlockSpec((BN, H), lambda t: (t, 0)),
            pl.BlockSpec((1, 1, BN), lambda t: (t, 0, 0)),
            pl.BlockSpec((H, H), lambda t: (0, 0)),
            pl.BlockSpec((1, H), lambda t: (0, 0)),
        ],
        out_specs=pl.BlockSpec((NG, H), lambda t: (0, 0)),
        out_shape=jax.ShapeDtypeStruct((NG, H), jnp.float32),
        interpret=interpret,
    )


def _row(b):
    return b.reshape(1, H)


def kernel(z, pos, edge_index, batch, emb, layers, W_ro, b_ro):
    ei = edge_index[0].astype(jnp.int32)
    ej = edge_index[1].astype(jnp.int32)

    rij = pos[ej] - pos[ei]
    dist = jnp.sqrt(jnp.sum(rij * rij, axis=-1))
    dist_safe = jnp.maximum(dist, EPS)
    unit = rij / dist_safe[:, None]
    ef = jnp.concatenate(
        [unit, dist_safe[:, None], jnp.zeros((N_E, 4), jnp.float32)], axis=1)
    ef = jnp.concatenate(
        [ef, jnp.tile(jnp.array([[0, 0, 0, 1, 0, 0, 0, 0]], jnp.float32),
                      (E_PAD - N_E, 1))], axis=0)
    ei2d = jnp.concatenate(
        [ei, jnp.full((E_PAD - N_E,), N_N, jnp.int32)]).reshape(NROWS, CH)
    ej2d = jnp.concatenate(
        [ej, jnp.zeros((E_PAD - N_E,), jnp.int32)]).reshape(NROWS, CH)

    s = jnp.zeros((N_PAD, H), jnp.float32).at[:N_N].set(emb[z])
    v0 = v1 = v2 = None

    gather1 = _sc_gather(1)
    gather4 = _sc_gather(4)
    scatter4 = _sc_scatter4()
    edge1 = _tc_edge(False)
    edge2 = _tc_edge(True)
    node = _tc_node()

    for li, p in enumerate(layers):
        w1t = jnp.zeros((H, H), jnp.float32).at[:NRBF, :].set(p['phi'][0].T)
        wargs = (w1t, _row(p['phi'][1]), p['phi'][2].T, _row(p['phi'][3]),
                 p['Ws_W'].T, _row(p['Ws_b']), p['Wv_W'].T, _row(p['Wv_b']))
        if li == 0:
            (sj,) = gather1(s, ej2d)
            ms_e, a0_e, a1_e, a2_e = edge1(ef, sj, *wargs)
        else:
            sj, vj0, vj1, vj2 = gather4(s, v0, v1, v2, ej2d)
            ms_e, a0_e, a1_e, a2_e = edge2(ef, sj, vj0, vj1, vj2, *wargs)
        MS, A0, A1, A2 = scatter4(ms_e, a0_e, a1_e, a2_e, ei2d)
        if li == 0:
            zz = jnp.zeros((N_PAD, H), jnp.float32)
            vin = (zz, zz, zz)
        else:
            vin = (v0, v1, v2)
        s, v0, v1, v2 = node(
            s, MS, *vin, A0, A1, A2,
            p['Us'][0].T, _row(p['Us'][1]), p['Us'][2].T, _row(p['Us'][3]),
            p['Uv'][0].T, _row(p['Uv'][1]), p['Uv'][2].T, _row(p['Uv'][3]))

    wrot = jnp.zeros((H, H), jnp.float32).at[:, :3].set(W_ro.T)
    brow = jnp.zeros((1, H), jnp.float32).at[0, :3].set(b_ro)
    batch3 = jnp.concatenate(
        [batch.astype(jnp.int32), jnp.full((N_PAD - N_N,), NG, jnp.int32)]
    ).reshape(N_PAD // BN, 1, BN)
    pred_pad = _tc_readout()(s, batch3, wrot, brow)
    return pred_pad[:, :3]
